# baseline (speedup 1.0000x reference)
import functools

import jax
import jax.numpy as jnp
from jax import lax
from jax.experimental import pallas as pl
from jax.experimental.pallas import tpu as pltpu
from jax.experimental.pallas import tpu_sc as plsc

N = 10000
E = 320000
D = 128

NC = 2
NS = 16
NW = NC * NS

MICRO = 128
NBUF = 2
SLABM = 8
N_PAD = 10112
ROWS_PER_SUB = N_PAD // NS
CHUNKS = [(i * MICRO, MICRO) for i in range(ROWS_PER_SUB // MICRO)]
if ROWS_PER_SUB % MICRO:
    CHUNKS.append((ROWS_PER_SUB - ROWS_PER_SUB % MICRO,
                   ROWS_PER_SUB % MICRO))
EPW = 10240
E_PAD = EPW * NW
NMICRO = EPW // MICRO
NOUTER = NMICRO // SLABM


def _make_sc_pass(with_deg):
    out_type = [jax.ShapeDtypeStruct((NC, N_PAD, D), jnp.float32)]
    if with_deg:
        out_type.append(jax.ShapeDtypeStruct((NC, N_PAD, D), jnp.float32))
    mesh = plsc.VectorSubcoreMesh(core_axis_name="c", subcore_axis_name="s")

    @functools.partial(
        pl.kernel, mesh=mesh, out_type=out_type,
        scratch_types=[
            pltpu.VMEM((SLABM, MICRO), jnp.int32),
            pltpu.VMEM((SLABM, MICRO), jnp.int32),
            pltpu.VMEM((NBUF, MICRO, D), jnp.float32),
            pltpu.VMEM_SHARED((N_PAD, D), jnp.float32),
            pltpu.SemaphoreType.DMA,
            pltpu.SemaphoreType.DMA,
            pltpu.SemaphoreType.DMA,
            pltpu.SemaphoreType.DMA,
        ])
    def sc_pass(gidx, sidx, table, zrow, ones_h, *refs):
        if with_deg:
            acc_out, deg_out, gi, si, rows, acc_sh = refs[:6]
            sems = refs[6:]
        else:
            acc_out, gi, si, rows, acc_sh = refs[:5]
            sems = refs[5:]
        gsem = sems[:2]
        ssem = sems[2:]
        c = lax.axis_index("c")
        s = lax.axis_index("s")
        w = c * NS + s
        zbase = s * ROWS_PER_SUB

        def zero_acc():
            pltpu.sync_copy(zrow, rows.at[0])
            for ofs, sz in CHUNKS:
                pltpu.sync_copy(rows.at[0, pl.ds(0, sz)],
                                acc_sh.at[pl.ds(zbase + ofs, sz)])

        def writeback(dst_hbm):
            for ofs, sz in CHUNKS:
                pltpu.sync_copy(acc_sh.at[pl.ds(zbase + ofs, sz)],
                                rows.at[0, pl.ds(0, sz)])
                pltpu.sync_copy(rows.at[0, pl.ds(0, sz)],
                                dst_hbm.at[c, pl.ds(zbase + ofs, sz)])

        def wait_scatter(b):
            pltpu.make_async_copy(rows.at[b], acc_sh.at[si.at[0]],
                                  ssem[b]).wait()

        zero_acc()
        pltpu.sync_copy(zrow, rows.at[1])
        plsc.subcore_barrier()
        base0 = w * NOUTER * SLABM
        pltpu.sync_copy(sidx.at[pl.ds(base0, SLABM)], si)
        pltpu.async_copy(rows.at[0], acc_sh.at[si.at[0]], ssem[0], add=True)
        pltpu.async_copy(rows.at[1], acc_sh.at[si.at[1]], ssem[1], add=True)

        def body(t, carry):
            wait_scatter(0)
            wait_scatter(1)
            base = (w * NOUTER + t) * SLABM
            pltpu.sync_copy(gidx.at[pl.ds(base, SLABM)], gi)
            pltpu.sync_copy(sidx.at[pl.ds(base, SLABM)], si)
            pltpu.async_copy(table.at[gi.at[0]], rows.at[0], gsem[0])
            for j in range(SLABM):
                b = j % 2
                pltpu.make_async_copy(table.at[gi.at[j]], rows.at[b],
                                      gsem[b]).wait()
                pltpu.async_copy(rows.at[b], acc_sh.at[si.at[j]],
                                 ssem[b], add=True)
                if j + 1 < SLABM:
                    if j >= 1:
                        wait_scatter(1 - b)
                    pltpu.async_copy(table.at[gi.at[j + 1]],
                                     rows.at[1 - b], gsem[1 - b])
            return carry

        lax.fori_loop(0, NOUTER, body, 0)
        wait_scatter(0)
        wait_scatter(1)
        plsc.subcore_barrier()
        writeback(acc_out)

        if with_deg:
            plsc.subcore_barrier()
            zero_acc()
            pltpu.sync_copy(zrow, rows.at[1])
            pltpu.sync_copy(ones_h, rows.at[0])
            plsc.subcore_barrier()
            pltpu.sync_copy(sidx.at[pl.ds(base0, SLABM)], si)
            for j in range(SLABM):
                pltpu.async_copy(rows.at[1], acc_sh.at[si.at[j]],
                                 ssem[j % 2], add=True)

            def dbody(t, carry):
                base = (w * NOUTER + t) * SLABM
                for p, buf in ((0, si), (1, gi)):
                    @pl.when(lax.rem(t, 2) == p)
                    def _():
                        pltpu.sync_copy(sidx.at[pl.ds(base, SLABM)], buf)
                        for j in range(SLABM):
                            pltpu.make_async_copy(
                                rows.at[0], acc_sh.at[buf.at[j]],
                                ssem[j % 2]).wait()
                        for j in range(SLABM):
                            pltpu.async_copy(rows.at[0],
                                             acc_sh.at[buf.at[j]],
                                             ssem[j % 2], add=True)
                return carry

            lax.fori_loop(0, NOUTER, dbody, 0)
            for j in range(SLABM):
                pltpu.make_async_copy(rows.at[0], acc_sh.at[si.at[0]],
                                      ssem[j % 2]).wait()
            plsc.subcore_barrier()
            writeback(deg_out)

    return sc_pass


_sc_pass_deg = _make_sc_pass(with_deg=True)
_sc_pass_plain = _make_sc_pass(with_deg=False)


BR = 1264


def _dense_body(p_ref, g_ref, x_ref, m1_ref, m2_ref, wl_ref, wr_ref,
                wo_ref, o_ref):
    agg = p_ref[0] + p_ref[1]
    deg = g_ref[0, :, 0:1] + g_ref[1, :, 0:1]
    mean = agg / jnp.maximum(deg, 1.0)
    cdims = (((1,), (1,)), ((), ()))
    h = lax.dot_general(mean, wl_ref[...], cdims,
                        preferred_element_type=jnp.float32)
    h = h + lax.dot_general(x_ref[...], wr_ref[...], cdims,
                            preferred_element_type=jnp.float32)
    h = jnp.maximum(h, 0.0) * m1_ref[...] * m2_ref[...]
    o_ref[...] = lax.dot_general(h, wo_ref[...], cdims,
                                 preferred_element_type=jnp.float32)


def _dense_stage(p, g, x, m1, m2, wl, wr, wo):
    return pl.pallas_call(
        _dense_body,
        grid=(N_PAD // BR,),
        in_specs=[
            pl.BlockSpec((NC, BR, D), lambda i: (0, i, 0)),
            pl.BlockSpec((NC, BR, D), lambda i: (0, i, 0)),
            pl.BlockSpec((BR, D), lambda i: (i, 0)),
            pl.BlockSpec((BR, D), lambda i: (i, 0)),
            pl.BlockSpec((BR, D), lambda i: (i, 0)),
            pl.BlockSpec((D, D), lambda i: (0, 0)),
            pl.BlockSpec((D, D), lambda i: (0, 0)),
            pl.BlockSpec((D, D), lambda i: (0, 0)),
        ],
        out_specs=pl.BlockSpec((BR, D), lambda i: (i, 0)),
        out_shape=jax.ShapeDtypeStruct((N_PAD, D), jnp.float32),
    )(p, g, x, m1, m2, wl, wr, wo)


CBR = 1000


def _combine_body(p_ref, o_ref):
    o_ref[...] = p_ref[0] + p_ref[1]


def _combine(p):
    return pl.pallas_call(
        _combine_body,
        grid=(N // CBR,),
        in_specs=[pl.BlockSpec((NC, CBR, D), lambda i: (0, i, 0))],
        out_specs=pl.BlockSpec((CBR, D), lambda i: (i, 0)),
        out_shape=jax.ShapeDtypeStruct((N, D), jnp.float32),
    )(p)


def kernel(features, edge_index, W_l, W_r, W_out, adj_mask1, adj_mask2):
    src = edge_index[0]
    dst = edge_index[1]
    pad = E_PAD - E
    padv = jnp.full((pad,), N, dtype=jnp.int32)
    src_p = jnp.concatenate([src, padv]).reshape(NW * NMICRO, MICRO)
    dst_p = jnp.concatenate([dst, padv]).reshape(NW * NMICRO, MICRO)

    rpad = N_PAD - N
    feat_pad = jnp.concatenate(
        [features, jnp.zeros((rpad, D), jnp.float32)], axis=0)
    m1_pad = jnp.concatenate(
        [adj_mask1, jnp.zeros((rpad, D), jnp.float32)], axis=0)
    m2_pad = jnp.concatenate(
        [adj_mask2, jnp.zeros((rpad, D), jnp.float32)], axis=0)

    zrow = jnp.zeros((MICRO, D), jnp.float32)
    ones_h = jnp.ones((MICRO, D), jnp.float32)

    acc1, deg1 = _sc_pass_deg(src_p, dst_p, feat_pad, zrow, ones_h)

    h2 = _dense_stage(acc1, deg1, feat_pad, m1_pad, m2_pad, W_l, W_r, W_out)

    (acc2,) = _sc_pass_plain(dst_p, src_p, h2, zrow, ones_h)

    return _combine(acc2)


# speedup vs baseline: 1.1050x; 1.1050x over previous
import functools

import jax
import jax.numpy as jnp
from jax import lax
from jax.experimental import pallas as pl
from jax.experimental.pallas import tpu as pltpu
from jax.experimental.pallas import tpu_sc as plsc

N = 10000
E = 320000
D = 128

NC = 2
NS = 16
NW = NC * NS

MICRO = 128
NBUF = 2
SLABM = 8
N_PAD = 10112
ROWS_PER_SUB = N_PAD // NS
CHUNKS = [(i * MICRO, MICRO) for i in range(ROWS_PER_SUB // MICRO)]
if ROWS_PER_SUB % MICRO:
    CHUNKS.append((ROWS_PER_SUB - ROWS_PER_SUB % MICRO,
                   ROWS_PER_SUB % MICRO))
EPW = 10240
E_PAD = EPW * NW
NMICRO = EPW // MICRO
NOUTER = NMICRO // SLABM
T0 = 13
T1 = 7


def _make_sc_pass(with_deg):
    out_type = [jax.ShapeDtypeStruct((NC, N_PAD, D), jnp.float32)]
    if with_deg:
        out_type.append(jax.ShapeDtypeStruct((NC, N_PAD, D), jnp.float32))
    mesh = plsc.VectorSubcoreMesh(core_axis_name="c", subcore_axis_name="s")

    @functools.partial(
        pl.kernel, mesh=mesh, out_type=out_type,
        scratch_types=[
            pltpu.VMEM((SLABM, MICRO), jnp.int32),
            pltpu.VMEM((SLABM, MICRO), jnp.int32),
            pltpu.VMEM((NBUF, MICRO, D), jnp.float32),
            pltpu.VMEM_SHARED((N_PAD, D), jnp.float32),
            pltpu.SemaphoreType.DMA,
            pltpu.SemaphoreType.DMA,
            pltpu.SemaphoreType.DMA,
            pltpu.SemaphoreType.DMA,
        ])
    def sc_pass(gidx, sidx, table, zrow, ones_h, *refs):
        if with_deg:
            acc_out, deg_out, gi, si, rows, acc_sh = refs[:6]
            sems = refs[6:]
        else:
            acc_out, gi, si, rows, acc_sh = refs[:5]
            sems = refs[5:]
        gsem = sems[:2]
        ssem = sems[2:]
        c = lax.axis_index("c")
        s = lax.axis_index("s")
        w = c * NS + s
        zbase = s * ROWS_PER_SUB

        def zero_acc():
            pltpu.sync_copy(zrow, rows.at[0])
            for ofs, sz in CHUNKS:
                pltpu.sync_copy(rows.at[0, pl.ds(0, sz)],
                                acc_sh.at[pl.ds(zbase + ofs, sz)])

        def writeback(dst_hbm):
            for ofs, sz in CHUNKS:
                pltpu.sync_copy(acc_sh.at[pl.ds(zbase + ofs, sz)],
                                rows.at[0, pl.ds(0, sz)])
                pltpu.sync_copy(rows.at[0, pl.ds(0, sz)],
                                dst_hbm.at[c, pl.ds(zbase + ofs, sz)])

        nslabs = jnp.where(c == 0, T0, T1)
        slab0 = jnp.where(c == 0, s * T0, NS * T0 + s * T1)

        zero_acc()
        plsc.subcore_barrier()

        def body(t, carry):
            base = (slab0 + t) * SLABM
            pltpu.sync_copy(gidx.at[pl.ds(base, SLABM)], gi)
            pltpu.sync_copy(sidx.at[pl.ds(base, SLABM)], si)
            pltpu.async_copy(table.at[gi.at[0]], rows.at[0], gsem[0])
            for j in range(SLABM):
                b = j % 2
                if j + 1 < SLABM:
                    pltpu.async_copy(table.at[gi.at[j + 1]],
                                     rows.at[1 - b], gsem[1 - b])
                pltpu.make_async_copy(table.at[gi.at[j]], rows.at[b],
                                      gsem[b]).wait()
                pltpu.sync_copy(rows.at[b], acc_sh.at[si.at[j]], add=True)
            return carry

        lax.fori_loop(0, nslabs, body, 0)
        plsc.subcore_barrier()
        writeback(acc_out)

        if with_deg:
            plsc.subcore_barrier()
            zero_acc()
            pltpu.sync_copy(ones_h, rows.at[0])
            plsc.subcore_barrier()

            def dbody(t, carry):
                base = (slab0 + t) * SLABM
                pltpu.sync_copy(sidx.at[pl.ds(base, SLABM)], si)
                for j in range(SLABM):
                    pltpu.sync_copy(rows.at[0], acc_sh.at[si.at[j]],
                                    add=True)
                return carry

            lax.fori_loop(0, nslabs, dbody, 0)
            plsc.subcore_barrier()
            writeback(deg_out)

    return sc_pass


_sc_pass_deg = _make_sc_pass(with_deg=True)
_sc_pass_plain = _make_sc_pass(with_deg=False)


BR = 1264


def _dense_body(p_ref, g_ref, x_ref, m1_ref, m2_ref, wl_ref, wr_ref,
                wo_ref, o_ref):
    agg = p_ref[0] + p_ref[1]
    deg = g_ref[0, :, 0:1] + g_ref[1, :, 0:1]
    mean = agg / jnp.maximum(deg, 1.0)
    cdims = (((1,), (1,)), ((), ()))
    h = lax.dot_general(mean, wl_ref[...], cdims,
                        preferred_element_type=jnp.float32)
    h = h + lax.dot_general(x_ref[...], wr_ref[...], cdims,
                            preferred_element_type=jnp.float32)
    h = jnp.maximum(h, 0.0) * m1_ref[...] * m2_ref[...]
    o_ref[...] = lax.dot_general(h, wo_ref[...], cdims,
                                 preferred_element_type=jnp.float32)


def _dense_stage(p, g, x, m1, m2, wl, wr, wo):
    return pl.pallas_call(
        _dense_body,
        grid=(N_PAD // BR,),
        in_specs=[
            pl.BlockSpec((NC, BR, D), lambda i: (0, i, 0)),
            pl.BlockSpec((NC, BR, D), lambda i: (0, i, 0)),
            pl.BlockSpec((BR, D), lambda i: (i, 0)),
            pl.BlockSpec((BR, D), lambda i: (i, 0)),
            pl.BlockSpec((BR, D), lambda i: (i, 0)),
            pl.BlockSpec((D, D), lambda i: (0, 0)),
            pl.BlockSpec((D, D), lambda i: (0, 0)),
            pl.BlockSpec((D, D), lambda i: (0, 0)),
        ],
        out_specs=pl.BlockSpec((BR, D), lambda i: (i, 0)),
        out_shape=jax.ShapeDtypeStruct((N_PAD, D), jnp.float32),
    )(p, g, x, m1, m2, wl, wr, wo)


CBR = 1000


def _combine_body(p_ref, o_ref):
    o_ref[...] = p_ref[0] + p_ref[1]


def _combine(p):
    return pl.pallas_call(
        _combine_body,
        grid=(N // CBR,),
        in_specs=[pl.BlockSpec((NC, CBR, D), lambda i: (0, i, 0))],
        out_specs=pl.BlockSpec((CBR, D), lambda i: (i, 0)),
        out_shape=jax.ShapeDtypeStruct((N, D), jnp.float32),
    )(p)


def kernel(features, edge_index, W_l, W_r, W_out, adj_mask1, adj_mask2):
    src = edge_index[0]
    dst = edge_index[1]
    pad = E_PAD - E
    padv = jnp.full((pad,), N, dtype=jnp.int32)
    src_p = jnp.concatenate([src, padv]).reshape(NW * NMICRO, MICRO)
    dst_p = jnp.concatenate([dst, padv]).reshape(NW * NMICRO, MICRO)

    rpad = N_PAD - N
    feat_pad = jnp.concatenate(
        [features, jnp.zeros((rpad, D), jnp.float32)], axis=0)
    m1_pad = jnp.concatenate(
        [adj_mask1, jnp.zeros((rpad, D), jnp.float32)], axis=0)
    m2_pad = jnp.concatenate(
        [adj_mask2, jnp.zeros((rpad, D), jnp.float32)], axis=0)

    zrow = jnp.zeros((MICRO, D), jnp.float32)
    ones_h = jnp.ones((MICRO, D), jnp.float32)

    acc1, deg1 = _sc_pass_deg(src_p, dst_p, feat_pad, zrow, ones_h)

    h2 = _dense_stage(acc1, deg1, feat_pad, m1_pad, m2_pad, W_l, W_r, W_out)

    (acc2,) = _sc_pass_plain(dst_p, src_p, h2, zrow, ones_h)

    return _combine(acc2)


# speedup vs baseline: 1.1553x; 1.0456x over previous
import functools

import jax
import jax.numpy as jnp
from jax import lax
from jax.experimental import pallas as pl
from jax.experimental.pallas import tpu as pltpu
from jax.experimental.pallas import tpu_sc as plsc

N = 10000
E = 320000
D = 128

NC = 2
NS = 16
NW = NC * NS

MICRO = 128
NBUF = 2
SLABM = 8
N_PAD = 10112
ROWS_PER_SUB = N_PAD // NS
CHUNKS = [(i * MICRO, MICRO) for i in range(ROWS_PER_SUB // MICRO)]
if ROWS_PER_SUB % MICRO:
    CHUNKS.append((ROWS_PER_SUB - ROWS_PER_SUB % MICRO,
                   ROWS_PER_SUB % MICRO))
EPW = 10240
E_PAD = EPW * NW
NMICRO = EPW // MICRO
NOUTER = NMICRO // SLABM
T0 = 15
T1 = 5


def _make_sc_pass(with_deg):
    out_type = [jax.ShapeDtypeStruct((NC, N_PAD, D), jnp.float32)]
    if with_deg:
        out_type.append(jax.ShapeDtypeStruct((NC, N_PAD, D), jnp.float32))
    mesh = plsc.VectorSubcoreMesh(core_axis_name="c", subcore_axis_name="s")

    @functools.partial(
        pl.kernel, mesh=mesh, out_type=out_type,
        scratch_types=[
            pltpu.VMEM((SLABM, MICRO), jnp.int32),
            pltpu.VMEM((SLABM, MICRO), jnp.int32),
            pltpu.VMEM((NBUF, MICRO, D), jnp.float32),
            pltpu.VMEM_SHARED((N_PAD, D), jnp.float32),
            pltpu.SemaphoreType.DMA,
            pltpu.SemaphoreType.DMA,
            pltpu.SemaphoreType.DMA,
            pltpu.SemaphoreType.DMA,
        ])
    def sc_pass(gidx, sidx, table, zrow, ones_h, *refs):
        if with_deg:
            acc_out, deg_out, gi, si, rows, acc_sh = refs[:6]
            sems = refs[6:]
        else:
            acc_out, gi, si, rows, acc_sh = refs[:5]
            sems = refs[5:]
        gsem = sems[:2]
        ssem = sems[2:]
        c = lax.axis_index("c")
        s = lax.axis_index("s")
        w = c * NS + s
        zbase = s * ROWS_PER_SUB

        def zero_acc():
            pltpu.sync_copy(zrow, rows.at[0])
            for ofs, sz in CHUNKS:
                pltpu.sync_copy(rows.at[0, pl.ds(0, sz)],
                                acc_sh.at[pl.ds(zbase + ofs, sz)])

        def writeback(dst_hbm):
            for ofs, sz in CHUNKS:
                pltpu.sync_copy(acc_sh.at[pl.ds(zbase + ofs, sz)],
                                rows.at[0, pl.ds(0, sz)])
                pltpu.sync_copy(rows.at[0, pl.ds(0, sz)],
                                dst_hbm.at[c, pl.ds(zbase + ofs, sz)])

        nslabs = jnp.where(c == 0, T0, T1)
        slab0 = jnp.where(c == 0, s * T0, NS * T0 + s * T1)

        zero_acc()
        plsc.subcore_barrier()

        def body(t, carry):
            base = (slab0 + t) * SLABM
            pltpu.sync_copy(gidx.at[pl.ds(base, SLABM)], gi)
            pltpu.sync_copy(sidx.at[pl.ds(base, SLABM)], si)
            pltpu.async_copy(table.at[gi.at[0]], rows.at[0], gsem[0])
            for j in range(SLABM):
                b = j % 2
                if j + 1 < SLABM:
                    pltpu.async_copy(table.at[gi.at[j + 1]],
                                     rows.at[1 - b], gsem[1 - b])
                pltpu.make_async_copy(table.at[gi.at[j]], rows.at[b],
                                      gsem[b]).wait()
                pltpu.sync_copy(rows.at[b], acc_sh.at[si.at[j]], add=True)
            return carry

        lax.fori_loop(0, nslabs, body, 0)
        plsc.subcore_barrier()
        writeback(acc_out)

        if with_deg:
            plsc.subcore_barrier()
            zero_acc()
            pltpu.sync_copy(ones_h, rows.at[0])
            plsc.subcore_barrier()

            def dbody(t, carry):
                base = (slab0 + t) * SLABM
                pltpu.sync_copy(sidx.at[pl.ds(base, SLABM)], si)
                for j in range(SLABM):
                    pltpu.sync_copy(rows.at[0], acc_sh.at[si.at[j]],
                                    add=True)
                return carry

            lax.fori_loop(0, nslabs, dbody, 0)
            plsc.subcore_barrier()
            writeback(deg_out)

    return sc_pass


_sc_pass_deg = _make_sc_pass(with_deg=True)
_sc_pass_plain = _make_sc_pass(with_deg=False)


BR = 1264


def _dense_body(p_ref, g_ref, x_ref, m1_ref, m2_ref, wl_ref, wr_ref,
                wo_ref, o_ref):
    agg = p_ref[0] + p_ref[1]
    deg = g_ref[0, :, 0:1] + g_ref[1, :, 0:1]
    mean = agg / jnp.maximum(deg, 1.0)
    cdims = (((1,), (1,)), ((), ()))
    h = lax.dot_general(mean, wl_ref[...], cdims,
                        preferred_element_type=jnp.float32)
    h = h + lax.dot_general(x_ref[...], wr_ref[...], cdims,
                            preferred_element_type=jnp.float32)
    h = jnp.maximum(h, 0.0) * m1_ref[...] * m2_ref[...]
    o_ref[...] = lax.dot_general(h, wo_ref[...], cdims,
                                 preferred_element_type=jnp.float32)


def _dense_stage(p, g, x, m1, m2, wl, wr, wo):
    return pl.pallas_call(
        _dense_body,
        grid=(N_PAD // BR,),
        in_specs=[
            pl.BlockSpec((NC, BR, D), lambda i: (0, i, 0)),
            pl.BlockSpec((NC, BR, D), lambda i: (0, i, 0)),
            pl.BlockSpec((BR, D), lambda i: (i, 0)),
            pl.BlockSpec((BR, D), lambda i: (i, 0)),
            pl.BlockSpec((BR, D), lambda i: (i, 0)),
            pl.BlockSpec((D, D), lambda i: (0, 0)),
            pl.BlockSpec((D, D), lambda i: (0, 0)),
            pl.BlockSpec((D, D), lambda i: (0, 0)),
        ],
        out_specs=pl.BlockSpec((BR, D), lambda i: (i, 0)),
        out_shape=jax.ShapeDtypeStruct((N_PAD, D), jnp.float32),
    )(p, g, x, m1, m2, wl, wr, wo)


CBR = 1000


def _combine_body(p_ref, o_ref):
    o_ref[...] = p_ref[0] + p_ref[1]


def _combine(p):
    return pl.pallas_call(
        _combine_body,
        grid=(N // CBR,),
        in_specs=[pl.BlockSpec((NC, CBR, D), lambda i: (0, i, 0))],
        out_specs=pl.BlockSpec((CBR, D), lambda i: (i, 0)),
        out_shape=jax.ShapeDtypeStruct((N, D), jnp.float32),
    )(p)


def kernel(features, edge_index, W_l, W_r, W_out, adj_mask1, adj_mask2):
    src = edge_index[0]
    dst = edge_index[1]
    pad = E_PAD - E
    padv = jnp.full((pad,), N, dtype=jnp.int32)
    src_p = jnp.concatenate([src, padv]).reshape(NW * NMICRO, MICRO)
    dst_p = jnp.concatenate([dst, padv]).reshape(NW * NMICRO, MICRO)

    rpad = N_PAD - N
    feat_pad = jnp.concatenate(
        [features, jnp.zeros((rpad, D), jnp.float32)], axis=0)
    m1_pad = jnp.concatenate(
        [adj_mask1, jnp.zeros((rpad, D), jnp.float32)], axis=0)
    m2_pad = jnp.concatenate(
        [adj_mask2, jnp.zeros((rpad, D), jnp.float32)], axis=0)

    zrow = jnp.zeros((MICRO, D), jnp.float32)
    ones_h = jnp.ones((MICRO, D), jnp.float32)

    acc1, deg1 = _sc_pass_deg(src_p, dst_p, feat_pad, zrow, ones_h)

    h2 = _dense_stage(acc1, deg1, feat_pad, m1_pad, m2_pad, W_l, W_r, W_out)

    (acc2,) = _sc_pass_plain(dst_p, src_p, h2, zrow, ones_h)

    return _combine(acc2)


# speedup vs baseline: 1.1814x; 1.0226x over previous
import functools

import jax
import jax.numpy as jnp
from jax import lax
from jax.experimental import pallas as pl
from jax.experimental.pallas import tpu as pltpu
from jax.experimental.pallas import tpu_sc as plsc

N = 10000
E = 320000
D = 128

NC = 2
NS = 16
NW = NC * NS

MICRO = 128
NBUF = 2
SLABM = 8
N_PAD = 10112
ROWS_PER_SUB = N_PAD // NS
CHUNKS = [(i * MICRO, MICRO) for i in range(ROWS_PER_SUB // MICRO)]
if ROWS_PER_SUB % MICRO:
    CHUNKS.append((ROWS_PER_SUB - ROWS_PER_SUB % MICRO,
                   ROWS_PER_SUB % MICRO))
EPW = 10240
E_PAD = EPW * NW
NMICRO = EPW // MICRO
NOUTER = NMICRO // SLABM
T0 = 16
T1 = 4


def _make_sc_pass(with_deg):
    out_type = [jax.ShapeDtypeStruct((NC, N_PAD, D), jnp.float32)]
    if with_deg:
        out_type.append(jax.ShapeDtypeStruct((NC, N_PAD, D), jnp.float32))
    mesh = plsc.VectorSubcoreMesh(core_axis_name="c", subcore_axis_name="s")

    @functools.partial(
        pl.kernel, mesh=mesh, out_type=out_type,
        scratch_types=[
            pltpu.VMEM((SLABM, MICRO), jnp.int32),
            pltpu.VMEM((SLABM, MICRO), jnp.int32),
            pltpu.VMEM((NBUF, MICRO, D), jnp.float32),
            pltpu.VMEM_SHARED((N_PAD, D), jnp.float32),
            pltpu.SemaphoreType.DMA,
            pltpu.SemaphoreType.DMA,
            pltpu.SemaphoreType.DMA,
            pltpu.SemaphoreType.DMA,
        ])
    def sc_pass(gidx, sidx, table, zrow, ones_h, *refs):
        if with_deg:
            acc_out, deg_out, gi, si, rows, acc_sh = refs[:6]
            sems = refs[6:]
        else:
            acc_out, gi, si, rows, acc_sh = refs[:5]
            sems = refs[5:]
        gsem = sems[:2]
        ssem = sems[2:]
        c = lax.axis_index("c")
        s = lax.axis_index("s")
        w = c * NS + s
        zbase = s * ROWS_PER_SUB

        def zero_acc():
            pltpu.sync_copy(zrow, rows.at[0])
            for ofs, sz in CHUNKS:
                pltpu.sync_copy(rows.at[0, pl.ds(0, sz)],
                                acc_sh.at[pl.ds(zbase + ofs, sz)])

        def writeback(dst_hbm):
            for ofs, sz in CHUNKS:
                pltpu.sync_copy(acc_sh.at[pl.ds(zbase + ofs, sz)],
                                rows.at[0, pl.ds(0, sz)])
                pltpu.sync_copy(rows.at[0, pl.ds(0, sz)],
                                dst_hbm.at[c, pl.ds(zbase + ofs, sz)])

        nslabs = jnp.where(c == 0, T0, T1)
        slab0 = jnp.where(c == 0, s * T0, NS * T0 + s * T1)

        zero_acc()
        plsc.subcore_barrier()

        def body(t, carry):
            base = (slab0 + t) * SLABM
            pltpu.sync_copy(gidx.at[pl.ds(base, SLABM)], gi)
            pltpu.sync_copy(sidx.at[pl.ds(base, SLABM)], si)
            pltpu.async_copy(table.at[gi.at[0]], rows.at[0], gsem[0])
            for j in range(SLABM):
                b = j % 2
                if j + 1 < SLABM:
                    pltpu.async_copy(table.at[gi.at[j + 1]],
                                     rows.at[1 - b], gsem[1 - b])
                pltpu.make_async_copy(table.at[gi.at[j]], rows.at[b],
                                      gsem[b]).wait()
                pltpu.sync_copy(rows.at[b], acc_sh.at[si.at[j]], add=True)
            return carry

        lax.fori_loop(0, nslabs, body, 0)
        plsc.subcore_barrier()
        writeback(acc_out)

        if with_deg:
            plsc.subcore_barrier()
            zero_acc()
            pltpu.sync_copy(ones_h, rows.at[0])
            plsc.subcore_barrier()

            def dbody(t, carry):
                base = (slab0 + t) * SLABM
                pltpu.sync_copy(sidx.at[pl.ds(base, SLABM)], si)
                for j in range(SLABM):
                    pltpu.sync_copy(rows.at[0], acc_sh.at[si.at[j]],
                                    add=True)
                return carry

            lax.fori_loop(0, nslabs, dbody, 0)
            plsc.subcore_barrier()
            writeback(deg_out)

    return sc_pass


_sc_pass_deg = _make_sc_pass(with_deg=True)
_sc_pass_plain = _make_sc_pass(with_deg=False)


BR = 1264


def _dense_body(p_ref, g_ref, x_ref, m1_ref, m2_ref, wl_ref, wr_ref,
                wo_ref, o_ref):
    agg = p_ref[0] + p_ref[1]
    deg = g_ref[0, :, 0:1] + g_ref[1, :, 0:1]
    mean = agg / jnp.maximum(deg, 1.0)
    cdims = (((1,), (1,)), ((), ()))
    h = lax.dot_general(mean, wl_ref[...], cdims,
                        preferred_element_type=jnp.float32)
    h = h + lax.dot_general(x_ref[...], wr_ref[...], cdims,
                            preferred_element_type=jnp.float32)
    h = jnp.maximum(h, 0.0) * m1_ref[...] * m2_ref[...]
    o_ref[...] = lax.dot_general(h, wo_ref[...], cdims,
                                 preferred_element_type=jnp.float32)


def _dense_stage(p, g, x, m1, m2, wl, wr, wo):
    return pl.pallas_call(
        _dense_body,
        grid=(N_PAD // BR,),
        in_specs=[
            pl.BlockSpec((NC, BR, D), lambda i: (0, i, 0)),
            pl.BlockSpec((NC, BR, D), lambda i: (0, i, 0)),
            pl.BlockSpec((BR, D), lambda i: (i, 0)),
            pl.BlockSpec((BR, D), lambda i: (i, 0)),
            pl.BlockSpec((BR, D), lambda i: (i, 0)),
            pl.BlockSpec((D, D), lambda i: (0, 0)),
            pl.BlockSpec((D, D), lambda i: (0, 0)),
            pl.BlockSpec((D, D), lambda i: (0, 0)),
        ],
        out_specs=pl.BlockSpec((BR, D), lambda i: (i, 0)),
        out_shape=jax.ShapeDtypeStruct((N_PAD, D), jnp.float32),
    )(p, g, x, m1, m2, wl, wr, wo)


CBR = 1000


def _combine_body(p_ref, o_ref):
    o_ref[...] = p_ref[0] + p_ref[1]


def _combine(p):
    return pl.pallas_call(
        _combine_body,
        grid=(N // CBR,),
        in_specs=[pl.BlockSpec((NC, CBR, D), lambda i: (0, i, 0))],
        out_specs=pl.BlockSpec((CBR, D), lambda i: (i, 0)),
        out_shape=jax.ShapeDtypeStruct((N, D), jnp.float32),
    )(p)


def kernel(features, edge_index, W_l, W_r, W_out, adj_mask1, adj_mask2):
    src = edge_index[0]
    dst = edge_index[1]
    pad = E_PAD - E
    padv = jnp.full((pad,), N, dtype=jnp.int32)
    src_p = jnp.concatenate([src, padv]).reshape(NW * NMICRO, MICRO)
    dst_p = jnp.concatenate([dst, padv]).reshape(NW * NMICRO, MICRO)

    rpad = N_PAD - N
    feat_pad = jnp.concatenate(
        [features, jnp.zeros((rpad, D), jnp.float32)], axis=0)
    m1_pad = jnp.concatenate(
        [adj_mask1, jnp.zeros((rpad, D), jnp.float32)], axis=0)
    m2_pad = jnp.concatenate(
        [adj_mask2, jnp.zeros((rpad, D), jnp.float32)], axis=0)

    zrow = jnp.zeros((MICRO, D), jnp.float32)
    ones_h = jnp.ones((MICRO, D), jnp.float32)

    acc1, deg1 = _sc_pass_deg(src_p, dst_p, feat_pad, zrow, ones_h)

    h2 = _dense_stage(acc1, deg1, feat_pad, m1_pad, m2_pad, W_l, W_r, W_out)

    (acc2,) = _sc_pass_plain(dst_p, src_p, h2, zrow, ones_h)

    return _combine(acc2)


# speedup vs baseline: 1.2100x; 1.0242x over previous
import functools

import jax
import jax.numpy as jnp
from jax import lax
from jax.experimental import pallas as pl
from jax.experimental.pallas import tpu as pltpu
from jax.experimental.pallas import tpu_sc as plsc

N = 10000
E = 320000
D = 128

NC = 2
NS = 16
NW = NC * NS

MICRO = 128
NBUF = 2
SLABM = 8
N_PAD = 10112
ROWS_PER_SUB = N_PAD // NS
CHUNKS = [(i * MICRO, MICRO) for i in range(ROWS_PER_SUB // MICRO)]
if ROWS_PER_SUB % MICRO:
    CHUNKS.append((ROWS_PER_SUB - ROWS_PER_SUB % MICRO,
                   ROWS_PER_SUB % MICRO))
EPW = 10240
E_PAD = EPW * NW
NMICRO = EPW // MICRO
NOUTER = NMICRO // SLABM
T0 = 17
T1 = 3


def _make_sc_pass(with_deg):
    out_type = [jax.ShapeDtypeStruct((NC, N_PAD, D), jnp.float32)]
    if with_deg:
        out_type.append(jax.ShapeDtypeStruct((NC, N_PAD, D), jnp.float32))
    mesh = plsc.VectorSubcoreMesh(core_axis_name="c", subcore_axis_name="s")

    @functools.partial(
        pl.kernel, mesh=mesh, out_type=out_type,
        scratch_types=[
            pltpu.VMEM((SLABM, MICRO), jnp.int32),
            pltpu.VMEM((SLABM, MICRO), jnp.int32),
            pltpu.VMEM((NBUF, MICRO, D), jnp.float32),
            pltpu.VMEM_SHARED((N_PAD, D), jnp.float32),
            pltpu.SemaphoreType.DMA,
            pltpu.SemaphoreType.DMA,
            pltpu.SemaphoreType.DMA,
            pltpu.SemaphoreType.DMA,
        ])
    def sc_pass(gidx, sidx, table, zrow, ones_h, *refs):
        if with_deg:
            acc_out, deg_out, gi, si, rows, acc_sh = refs[:6]
            sems = refs[6:]
        else:
            acc_out, gi, si, rows, acc_sh = refs[:5]
            sems = refs[5:]
        gsem = sems[:2]
        ssem = sems[2:]
        c = lax.axis_index("c")
        s = lax.axis_index("s")
        w = c * NS + s
        zbase = s * ROWS_PER_SUB

        def zero_acc():
            pltpu.sync_copy(zrow, rows.at[0])
            for ofs, sz in CHUNKS:
                pltpu.sync_copy(rows.at[0, pl.ds(0, sz)],
                                acc_sh.at[pl.ds(zbase + ofs, sz)])

        def writeback(dst_hbm):
            for ofs, sz in CHUNKS:
                pltpu.sync_copy(acc_sh.at[pl.ds(zbase + ofs, sz)],
                                rows.at[0, pl.ds(0, sz)])
                pltpu.sync_copy(rows.at[0, pl.ds(0, sz)],
                                dst_hbm.at[c, pl.ds(zbase + ofs, sz)])

        nslabs = jnp.where(c == 0, T0, T1)
        slab0 = jnp.where(c == 0, s * T0, NS * T0 + s * T1)

        zero_acc()
        plsc.subcore_barrier()

        def body(t, carry):
            base = (slab0 + t) * SLABM
            pltpu.sync_copy(gidx.at[pl.ds(base, SLABM)], gi)
            pltpu.sync_copy(sidx.at[pl.ds(base, SLABM)], si)
            pltpu.async_copy(table.at[gi.at[0]], rows.at[0], gsem[0])
            for j in range(SLABM):
                b = j % 2
                if j + 1 < SLABM:
                    pltpu.async_copy(table.at[gi.at[j + 1]],
                                     rows.at[1 - b], gsem[1 - b])
                pltpu.make_async_copy(table.at[gi.at[j]], rows.at[b],
                                      gsem[b]).wait()
                pltpu.sync_copy(rows.at[b], acc_sh.at[si.at[j]], add=True)
            return carry

        lax.fori_loop(0, nslabs, body, 0)
        plsc.subcore_barrier()
        writeback(acc_out)

        if with_deg:
            plsc.subcore_barrier()
            zero_acc()
            pltpu.sync_copy(ones_h, rows.at[0])
            plsc.subcore_barrier()

            def dbody(t, carry):
                base = (slab0 + t) * SLABM
                pltpu.sync_copy(sidx.at[pl.ds(base, SLABM)], si)
                for j in range(SLABM):
                    pltpu.sync_copy(rows.at[0], acc_sh.at[si.at[j]],
                                    add=True)
                return carry

            lax.fori_loop(0, nslabs, dbody, 0)
            plsc.subcore_barrier()
            writeback(deg_out)

    return sc_pass


_sc_pass_deg = _make_sc_pass(with_deg=True)
_sc_pass_plain = _make_sc_pass(with_deg=False)


BR = 1264


def _dense_body(p_ref, g_ref, x_ref, m1_ref, m2_ref, wl_ref, wr_ref,
                wo_ref, o_ref):
    agg = p_ref[0] + p_ref[1]
    deg = g_ref[0, :, 0:1] + g_ref[1, :, 0:1]
    mean = agg / jnp.maximum(deg, 1.0)
    cdims = (((1,), (1,)), ((), ()))
    h = lax.dot_general(mean, wl_ref[...], cdims,
                        preferred_element_type=jnp.float32)
    h = h + lax.dot_general(x_ref[...], wr_ref[...], cdims,
                            preferred_element_type=jnp.float32)
    h = jnp.maximum(h, 0.0) * m1_ref[...] * m2_ref[...]
    o_ref[...] = lax.dot_general(h, wo_ref[...], cdims,
                                 preferred_element_type=jnp.float32)


def _dense_stage(p, g, x, m1, m2, wl, wr, wo):
    return pl.pallas_call(
        _dense_body,
        grid=(N_PAD // BR,),
        in_specs=[
            pl.BlockSpec((NC, BR, D), lambda i: (0, i, 0)),
            pl.BlockSpec((NC, BR, D), lambda i: (0, i, 0)),
            pl.BlockSpec((BR, D), lambda i: (i, 0)),
            pl.BlockSpec((BR, D), lambda i: (i, 0)),
            pl.BlockSpec((BR, D), lambda i: (i, 0)),
            pl.BlockSpec((D, D), lambda i: (0, 0)),
            pl.BlockSpec((D, D), lambda i: (0, 0)),
            pl.BlockSpec((D, D), lambda i: (0, 0)),
        ],
        out_specs=pl.BlockSpec((BR, D), lambda i: (i, 0)),
        out_shape=jax.ShapeDtypeStruct((N_PAD, D), jnp.float32),
    )(p, g, x, m1, m2, wl, wr, wo)


CBR = 1000


def _combine_body(p_ref, o_ref):
    o_ref[...] = p_ref[0] + p_ref[1]


def _combine(p):
    return pl.pallas_call(
        _combine_body,
        grid=(N // CBR,),
        in_specs=[pl.BlockSpec((NC, CBR, D), lambda i: (0, i, 0))],
        out_specs=pl.BlockSpec((CBR, D), lambda i: (i, 0)),
        out_shape=jax.ShapeDtypeStruct((N, D), jnp.float32),
    )(p)


def kernel(features, edge_index, W_l, W_r, W_out, adj_mask1, adj_mask2):
    src = edge_index[0]
    dst = edge_index[1]
    pad = E_PAD - E
    padv = jnp.full((pad,), N, dtype=jnp.int32)
    src_p = jnp.concatenate([src, padv]).reshape(NW * NMICRO, MICRO)
    dst_p = jnp.concatenate([dst, padv]).reshape(NW * NMICRO, MICRO)

    rpad = N_PAD - N
    feat_pad = jnp.concatenate(
        [features, jnp.zeros((rpad, D), jnp.float32)], axis=0)
    m1_pad = jnp.concatenate(
        [adj_mask1, jnp.zeros((rpad, D), jnp.float32)], axis=0)
    m2_pad = jnp.concatenate(
        [adj_mask2, jnp.zeros((rpad, D), jnp.float32)], axis=0)

    zrow = jnp.zeros((MICRO, D), jnp.float32)
    ones_h = jnp.ones((MICRO, D), jnp.float32)

    acc1, deg1 = _sc_pass_deg(src_p, dst_p, feat_pad, zrow, ones_h)

    h2 = _dense_stage(acc1, deg1, feat_pad, m1_pad, m2_pad, W_l, W_r, W_out)

    (acc2,) = _sc_pass_plain(dst_p, src_p, h2, zrow, ones_h)

    return _combine(acc2)
